# Initial kernel scaffold; baseline (speedup 1.0000x reference)
import jax
import jax.numpy as jnp
from jax.experimental import pallas as pl


def kernel(input1, input2, emb_table, fc_w, fc_b):
    raise NotImplementedError("write your pallas kernel here")


# speedup vs baseline: 5.1271x; 5.1271x over previous
import jax
import jax.numpy as jnp
from jax import lax
from jax.experimental import pallas as pl
from jax.experimental.pallas import tpu as pltpu
from jax.experimental.pallas import tpu_sc as plsc

_NUM_EMB = 1000000
_D = 128
_B = 16384
_NC, _NS = 2, 16
_NW = _NC * _NS
_BPW = _B // _NW
_CH = 128
_NCH = _BPW // _CH
_NG = _CH // 16


def _sc_body(idx1_hbm, idx2_hbm, table_hbm, w_hbm, out_hbm,
             idx1_v, idx2_v, w_v, rows1_v, rows2_v, out_v, sem1, sem2):
    wid = lax.axis_index("s") * _NC + lax.axis_index("c")
    r0 = wid * _NCH
    pltpu.sync_copy(idx1_hbm.at[pl.ds(r0, _NCH)], idx1_v)
    pltpu.sync_copy(idx2_hbm.at[pl.ds(r0, _NCH)], idx2_v)
    pltpu.sync_copy(w_hbm, w_v)

    lane = lax.iota(jnp.int32, 16)
    zeros16 = jnp.zeros((16,), jnp.int32)
    bias = w_v[pl.ds(256, 16)].at[zeros16].get(mode="promise_in_bounds")

    def chunk_body(c, carry):
        pltpu.async_copy(table_hbm.at[idx1_v.at[c]], rows1_v, sem1).wait()
        pltpu.async_copy(table_hbm.at[idx2_v.at[c]], rows2_v, sem2).wait()

        def group_body(g, gcarry):
            rid = lane + g * 16
            acc = bias
            for d in range(_D):
                k, j = d // 16, d % 16
                jvec = jnp.full((16,), j, jnp.int32)
                dvec = jnp.full((16,), d, jnp.int32)
                w1 = w_v[pl.ds(k * 16, 16)].at[jvec].get(
                    mode="promise_in_bounds")
                w2 = w_v[pl.ds(128 + k * 16, 16)].at[jvec].get(
                    mode="promise_in_bounds")
                col1 = plsc.load_gather(rows1_v, [rid, dvec])
                col2 = plsc.load_gather(rows2_v, [rid, dvec])
                acc = acc + col1 * w1 + col2 * w2
            sig = 1.0 / (1.0 + jnp.exp(-acc))
            out_v[pl.ds(c * _CH + g * 16, 16)] = sig
            return gcarry

        lax.fori_loop(0, _NG, group_body, 0)
        return carry

    lax.fori_loop(0, _NCH, chunk_body, 0)
    pltpu.sync_copy(out_v, out_hbm.at[pl.ds(wid * _BPW, _BPW)])


@jax.jit
def kernel(input1, input2, emb_table, fc_w, fc_b):
    idx1 = input1.astype(jnp.int32).reshape(_NW * _NCH, _CH)
    idx2 = input2.astype(jnp.int32).reshape(_NW * _NCH, _CH)
    w = jnp.concatenate(
        [fc_w.reshape(-1), fc_b.reshape(-1),
         jnp.zeros((15,), jnp.float32)]).astype(jnp.float32)
    mesh = plsc.VectorSubcoreMesh(core_axis_name="c", subcore_axis_name="s",
                                  num_cores=_NC, num_subcores=_NS)
    out = pl.kernel(
        _sc_body,
        out_type=jax.ShapeDtypeStruct((_B,), jnp.float32),
        mesh=mesh,
        compiler_params=pltpu.CompilerParams(needs_layout_passes=False),
        scratch_types=[
            pltpu.VMEM((_NCH, _CH), jnp.int32),
            pltpu.VMEM((_NCH, _CH), jnp.int32),
            pltpu.VMEM((272,), jnp.float32),
            pltpu.VMEM((_CH, _D), jnp.float32),
            pltpu.VMEM((_CH, _D), jnp.float32),
            pltpu.VMEM((_BPW,), jnp.float32),
            pltpu.SemaphoreType.DMA,
            pltpu.SemaphoreType.DMA,
        ],
    )(idx1, idx2, emb_table, w)
    return out.reshape(_B, 1)


# speedup vs baseline: 5.2836x; 1.0305x over previous
import jax
import jax.numpy as jnp
from jax import lax
from jax.experimental import pallas as pl
from jax.experimental.pallas import tpu as pltpu
from jax.experimental.pallas import tpu_sc as plsc

_NUM_EMB = 1000000
_D = 128
_B = 16384
_NC, _NS = 2, 16
_NW = _NC * _NS
_BPW = _B // _NW
_CH = 128
_NCH = _BPW // _CH
_NG = _CH // 16


def _sc_body(idx1_hbm, idx2_hbm, table_hbm, w_hbm, out_hbm,
             idx1_v, idx2_v, w_v, rows1_v, rows2_v, out_v,
             sem1a, sem1b, sem2a, sem2b):
    wid = lax.axis_index("s") * _NC + lax.axis_index("c")
    r0 = wid * _NCH
    pltpu.sync_copy(idx1_hbm.at[pl.ds(r0, _NCH)], idx1_v)
    pltpu.sync_copy(idx2_hbm.at[pl.ds(r0, _NCH)], idx2_v)
    pltpu.sync_copy(w_hbm, w_v)

    sems1 = (sem1a, sem1b)
    sems2 = (sem2a, sem2b)

    def issue(c):
        slot = c % 2
        d1 = pltpu.async_copy(table_hbm.at[idx1_v.at[c]],
                              rows1_v.at[slot], sems1[slot])
        d2 = pltpu.async_copy(table_hbm.at[idx2_v.at[c]],
                              rows2_v.at[slot], sems2[slot])
        return d1, d2

    lane = lax.iota(jnp.int32, 16)
    zeros16 = jnp.zeros((16,), jnp.int32)
    bias = w_v[pl.ds(256, 16)].at[zeros16].get(mode="promise_in_bounds")

    pending = issue(0)
    for c in range(_NCH):
        slot = c % 2
        d1, d2 = pending
        if c + 1 < _NCH:
            pending = issue(c + 1)
        d1.wait()
        d2.wait()

        def group_body(g, gcarry, slot=slot, c=c):
            rid = lane + g * 16
            acc = bias
            for d in range(_D):
                k, j = d // 16, d % 16
                jvec = jnp.full((16,), j, jnp.int32)
                dvec = jnp.full((16,), d, jnp.int32)
                w1 = w_v[pl.ds(k * 16, 16)].at[jvec].get(
                    mode="promise_in_bounds")
                w2 = w_v[pl.ds(128 + k * 16, 16)].at[jvec].get(
                    mode="promise_in_bounds")
                col1 = plsc.load_gather(rows1_v.at[slot], [rid, dvec])
                col2 = plsc.load_gather(rows2_v.at[slot], [rid, dvec])
                acc = acc + col1 * w1 + col2 * w2
            sig = 1.0 / (1.0 + jnp.exp(-acc))
            out_v[pl.ds(c * _CH + g * 16, 16)] = sig
            return gcarry

        lax.fori_loop(0, _NG, group_body, 0)
    pltpu.sync_copy(out_v, out_hbm.at[pl.ds(wid * _BPW, _BPW)])


@jax.jit
def kernel(input1, input2, emb_table, fc_w, fc_b):
    idx1 = input1.astype(jnp.int32).reshape(_NW * _NCH, _CH)
    idx2 = input2.astype(jnp.int32).reshape(_NW * _NCH, _CH)
    w = jnp.concatenate(
        [fc_w.reshape(-1), fc_b.reshape(-1),
         jnp.zeros((15,), jnp.float32)]).astype(jnp.float32)
    mesh = plsc.VectorSubcoreMesh(core_axis_name="c", subcore_axis_name="s",
                                  num_cores=_NC, num_subcores=_NS)
    out = pl.kernel(
        _sc_body,
        out_type=jax.ShapeDtypeStruct((_B,), jnp.float32),
        mesh=mesh,
        compiler_params=pltpu.CompilerParams(needs_layout_passes=False),
        scratch_types=[
            pltpu.VMEM((_NCH, _CH), jnp.int32),
            pltpu.VMEM((_NCH, _CH), jnp.int32),
            pltpu.VMEM((272,), jnp.float32),
            pltpu.VMEM((2, _CH, _D), jnp.float32),
            pltpu.VMEM((2, _CH, _D), jnp.float32),
            pltpu.VMEM((_BPW,), jnp.float32),
            pltpu.SemaphoreType.DMA,
            pltpu.SemaphoreType.DMA,
            pltpu.SemaphoreType.DMA,
            pltpu.SemaphoreType.DMA,
        ],
    )(idx1, idx2, emb_table, w)
    return out.reshape(_B, 1)


# speedup vs baseline: 12.9578x; 2.4525x over previous
import jax
import jax.numpy as jnp
from jax import lax
from jax.experimental import pallas as pl
from jax.experimental.pallas import tpu as pltpu
from jax.experimental.pallas import tpu_sc as plsc

_NUM_EMB = 1000000
_D = 128
_B = 16384
_NC, _NS = 2, 16
_NW = _NC * _NS
_BPW = _B // _NW
_CH = 128
_NCH = _BPW // _CH
_NG = _CH // 16
_BITREV = [0, 8, 4, 12, 2, 10, 6, 14, 1, 9, 5, 13, 3, 11, 7, 15]


def _sc_body(idx1_hbm, idx2_hbm, table_hbm, w_hbm, out_hbm,
             idx1_v, idx2_v, w_v, rows1_v, rows2_v, out_v,
             sem1a, sem1b, sem2a, sem2b):
    wid = lax.axis_index("s") * _NC + lax.axis_index("c")
    r0 = wid * _NCH
    pltpu.sync_copy(idx1_hbm.at[pl.ds(r0, _NCH)], idx1_v)
    pltpu.sync_copy(idx2_hbm.at[pl.ds(r0, _NCH)], idx2_v)
    pltpu.sync_copy(w_hbm, w_v)

    sems1 = (sem1a, sem1b)
    sems2 = (sem2a, sem2b)

    def issue(c):
        slot = c % 2
        d1 = pltpu.async_copy(table_hbm.at[idx1_v.at[c]],
                              rows1_v.at[slot], sems1[slot])
        d2 = pltpu.async_copy(table_hbm.at[idx2_v.at[c]],
                              rows2_v.at[slot], sems2[slot])
        return d1, d2

    lane = lax.iota(jnp.int32, 16)
    zeros16 = jnp.zeros((16,), jnp.int32)
    bias = w_v[pl.ds(256, 16)].at[zeros16].get(mode="promise_in_bounds")
    w1k = [w_v[pl.ds(k * 16, 16)] for k in range(8)]
    w2k = [w_v[pl.ds(128 + k * 16, 16)] for k in range(8)]
    maskv = {m: (lane & m) == 0 for m in (8, 4, 2, 1)}
    permv = {m: lane ^ m for m in (8, 4, 2, 1)}

    def combine(a, b, m):
        t1 = jnp.where(maskv[m], a, b)
        t2 = jnp.where(maskv[m], b, a).at[permv[m]].get(
            mode="promise_in_bounds")
        return t1 + t2

    pending = issue(0)
    for c in range(_NCH):
        slot = c % 2
        d1, d2 = pending
        if c + 1 < _NCH:
            pending = issue(c + 1)
        d1.wait()
        d2.wait()

        def group_body(g, gcarry, slot=slot, c=c):
            base = g * 16

            def row_acc(j):
                r = base + _BITREV[j]
                rowv1 = rows1_v.at[slot].at[r]
                rowv2 = rows2_v.at[slot].at[r]
                acc = rowv1[pl.ds(0, 16)] * w1k[0]
                for k in range(1, 8):
                    acc = acc + rowv1[pl.ds(k * 16, 16)] * w1k[k]
                for k in range(8):
                    acc = acc + rowv2[pl.ds(k * 16, 16)] * w2k[k]
                return acc

            vs = [row_acc(j) for j in range(16)]
            for m in (8, 4, 2, 1):
                vs = [combine(vs[2 * i], vs[2 * i + 1], m)
                      for i in range(len(vs) // 2)]
            z = vs[0] + bias
            sig = 1.0 / (1.0 + jnp.exp(-z))
            out_v[pl.ds(c * _CH + base, 16)] = sig
            return gcarry

        lax.fori_loop(0, _NG, group_body, 0)
    pltpu.sync_copy(out_v, out_hbm.at[pl.ds(wid * _BPW, _BPW)])


@jax.jit
def kernel(input1, input2, emb_table, fc_w, fc_b):
    idx1 = input1.astype(jnp.int32).reshape(_NW * _NCH, _CH)
    idx2 = input2.astype(jnp.int32).reshape(_NW * _NCH, _CH)
    w = jnp.concatenate(
        [fc_w.reshape(-1), fc_b.reshape(-1),
         jnp.zeros((15,), jnp.float32)]).astype(jnp.float32)
    mesh = plsc.VectorSubcoreMesh(core_axis_name="c", subcore_axis_name="s",
                                  num_cores=_NC, num_subcores=_NS)
    out = pl.kernel(
        _sc_body,
        out_type=jax.ShapeDtypeStruct((_B,), jnp.float32),
        mesh=mesh,
        compiler_params=pltpu.CompilerParams(needs_layout_passes=False),
        scratch_types=[
            pltpu.VMEM((_NCH, _CH), jnp.int32),
            pltpu.VMEM((_NCH, _CH), jnp.int32),
            pltpu.VMEM((272,), jnp.float32),
            pltpu.VMEM((2, _CH, _D), jnp.float32),
            pltpu.VMEM((2, _CH, _D), jnp.float32),
            pltpu.VMEM((_BPW,), jnp.float32),
            pltpu.SemaphoreType.DMA,
            pltpu.SemaphoreType.DMA,
            pltpu.SemaphoreType.DMA,
            pltpu.SemaphoreType.DMA,
        ],
    )(idx1, idx2, emb_table, w)
    return out.reshape(_B, 1)


# speedup vs baseline: 13.3574x; 1.0308x over previous
import jax
import jax.numpy as jnp
from jax import lax
from jax.experimental import pallas as pl
from jax.experimental.pallas import tpu as pltpu
from jax.experimental.pallas import tpu_sc as plsc

_NUM_EMB = 1000000
_D = 128
_B = 16384
_NC, _NS = 2, 16
_NW = _NC * _NS
_BPW = _B // _NW
_CH = 128
_NCH = _BPW // _CH
_NG = _CH // 16
_BITREV = [0, 8, 4, 12, 2, 10, 6, 14, 1, 9, 5, 13, 3, 11, 7, 15]


def _sc_body(idx1_hbm, idx2_hbm, table_hbm, w_hbm, out_hbm,
             idx1_v, idx2_v, w_v, rows1_v, rows2_v, out_v,
             sem1a, sem1b, sem2a, sem2b):
    wid = lax.axis_index("s") * _NC + lax.axis_index("c")
    r0 = wid * _NCH
    pltpu.sync_copy(idx1_hbm.at[pl.ds(r0, _NCH)], idx1_v)
    pltpu.sync_copy(idx2_hbm.at[pl.ds(r0, _NCH)], idx2_v)
    pltpu.sync_copy(w_hbm, w_v)

    sems1 = (sem1a, sem1b)
    sems2 = (sem2a, sem2b)

    def issue(c):
        slot = c % 2
        d1 = pltpu.async_copy(table_hbm.at[idx1_v.at[c]],
                              rows1_v.at[slot], sems1[slot])
        d2 = pltpu.async_copy(table_hbm.at[idx2_v.at[c]],
                              rows2_v.at[slot], sems2[slot])
        return d1, d2

    lane = lax.iota(jnp.int32, 16)
    zeros16 = jnp.zeros((16,), jnp.int32)
    bias = w_v[pl.ds(256, 16)].at[zeros16].get(mode="promise_in_bounds")
    maskv = {m: (lane & m) == 0 for m in (8, 4, 2, 1)}
    permv = {m: lane ^ m for m in (8, 4, 2, 1)}

    def combine(a, b, m):
        t1 = jnp.where(maskv[m], a, b)
        t2 = jnp.where(maskv[m], b, a).at[permv[m]].get(
            mode="promise_in_bounds")
        return t1 + t2

    pending = issue(0)
    for c in range(_NCH):
        slot = c % 2
        d1, d2 = pending
        if c + 1 < _NCH:
            pending = issue(c + 1)
        d1.wait()
        d2.wait()

        def group_body(g, gcarry, slot=slot, c=c):
            base = g * 16
            mstage = (8, 4, 2, 1)
            stack = []

            for half in range(2):
                accs = [None] * 8
                for k in range(16):
                    if k < 8:
                        w = w_v[pl.ds(k * 16, 16)]
                    else:
                        w = w_v[pl.ds(128 + (k - 8) * 16, 16)]
                    for i in range(8):
                        r = base + _BITREV[half * 8 + i]
                        src = rows1_v if k < 8 else rows2_v
                        p = src.at[slot].at[r][pl.ds((k % 8) * 16, 16)] * w
                        accs[i] = p if accs[i] is None else accs[i] + p
                for i in range(8):
                    lvl, v = 0, accs[i]
                    while stack and stack[-1][0] == lvl:
                        _, pv = stack.pop()
                        v = combine(pv, v, mstage[lvl])
                        lvl += 1
                    stack.append((lvl, v))
            z = stack[0][1] + bias
            sig = 1.0 / (1.0 + jnp.exp(-z))
            out_v[pl.ds(c * _CH + base, 16)] = sig
            return gcarry

        lax.fori_loop(0, _NG, group_body, 0)
    pltpu.sync_copy(out_v, out_hbm.at[pl.ds(wid * _BPW, _BPW)])


@jax.jit
def kernel(input1, input2, emb_table, fc_w, fc_b):
    idx1 = input1.astype(jnp.int32).reshape(_NW * _NCH, _CH)
    idx2 = input2.astype(jnp.int32).reshape(_NW * _NCH, _CH)
    w = jnp.concatenate(
        [fc_w.reshape(-1), fc_b.reshape(-1),
         jnp.zeros((15,), jnp.float32)]).astype(jnp.float32)
    mesh = plsc.VectorSubcoreMesh(core_axis_name="c", subcore_axis_name="s",
                                  num_cores=_NC, num_subcores=_NS)
    out = pl.kernel(
        _sc_body,
        out_type=jax.ShapeDtypeStruct((_B,), jnp.float32),
        mesh=mesh,
        compiler_params=pltpu.CompilerParams(needs_layout_passes=False),
        scratch_types=[
            pltpu.VMEM((_NCH, _CH), jnp.int32),
            pltpu.VMEM((_NCH, _CH), jnp.int32),
            pltpu.VMEM((272,), jnp.float32),
            pltpu.VMEM((2, _CH, _D), jnp.float32),
            pltpu.VMEM((2, _CH, _D), jnp.float32),
            pltpu.VMEM((_BPW,), jnp.float32),
            pltpu.SemaphoreType.DMA,
            pltpu.SemaphoreType.DMA,
            pltpu.SemaphoreType.DMA,
            pltpu.SemaphoreType.DMA,
        ],
    )(idx1, idx2, emb_table, w)
    return out.reshape(_B, 1)
